# baseline (speedup 1.0000x reference)
import jax
import jax.numpy as jnp
from jax.experimental import pallas as pl
from jax.experimental.pallas import tpu as pltpu


def _expert_copies(e, s, w1_hbm, b1_hbm, w2_hbm, b2_hbm,
                   w1buf, b1buf, w2buf, b2buf, copy_sems):
    F = w1_hbm.shape[1] // 2
    half = w2_hbm.shape[1] // 2
    return (
        pltpu.make_async_copy(w1_hbm.at[e, pl.ds(0, F)],
                              w1buf.at[s, pl.ds(0, F)], copy_sems.at[s, 0]),
        pltpu.make_async_copy(w1_hbm.at[e, pl.ds(F, F)],
                              w1buf.at[s, pl.ds(F, F)], copy_sems.at[s, 1]),
        pltpu.make_async_copy(w2_hbm.at[e, pl.ds(0, half)],
                              w2buf.at[s, pl.ds(0, half)], copy_sems.at[s, 2]),
        pltpu.make_async_copy(w2_hbm.at[e, pl.ds(half, half)],
                              w2buf.at[s, pl.ds(half, half)], copy_sems.at[s, 3]),
        pltpu.make_async_copy(b1_hbm.at[e], b1buf.at[s], copy_sems.at[s, 4]),
        pltpu.make_async_copy(b2_hbm.at[e], b2buf.at[s], copy_sems.at[s, 5]),
    )


def _start_expert_copies(e, s, *rest):
    for c in _expert_copies(e, s, *rest):
        c.start()


def _body(E, F, B, D,
          x_ref, scale_ref, gw_ref, gb_ref, w1_hbm, b1_hbm, w2_hbm, b2_hbm,
          out_ref,
          t_scr, w_scr, meta_vmem, meta_smem,
          w1buf, b1buf, w2buf, b2buf, meta_sem, copy_sems):
    bufs = (w1_hbm, b1_hbm, w2_hbm, b2_hbm, w1buf, b1buf, w2buf, b2buf,
            copy_sems)
    half = D // 2

    x = x_ref[...]
    t = x * jax.lax.rsqrt(jnp.mean(x * x, axis=-1, keepdims=True) + 1e-5)
    t = t * scale_ref[...]
    t_scr[...] = t
    g = jax.lax.dot_general(
        t, gw_ref[...], (((1,), (1,)), ((), ())),
        preferred_element_type=jnp.float32) + gb_ref[...]

    col = jax.lax.broadcasted_iota(jnp.int32, g.shape, 1)
    v1 = jnp.max(g, axis=-1, keepdims=True)
    e1 = jnp.min(jnp.where(g == v1, col, E), axis=-1, keepdims=True)
    first1 = (col == e1)
    g2 = jnp.where(first1, -1e30, g)
    v2 = jnp.max(g2, axis=-1, keepdims=True)
    e2 = jnp.min(jnp.where(g2 == v2, col, E), axis=-1, keepdims=True)
    first2 = (col == e2)
    p1 = jax.nn.sigmoid(v1 - v2)
    wmat = jnp.where(first1, p1, 0.0) + jnp.where(first2, 1.0 - p1, 0.0)
    w_scr[...] = wmat

    act = (jnp.max(wmat, axis=0, keepdims=True) > 0.0)
    coli = jax.lax.broadcasted_iota(jnp.int32, (1, E), 1)
    e0v = jnp.min(jnp.where(act, coli, E), axis=1, keepdims=True)
    meta_vmem[...] = jnp.broadcast_to(e0v, (E + 1, 1)).astype(jnp.int32)
    pltpu.make_async_copy(meta_vmem, meta_smem, meta_sem).start()
    pltpu.make_async_copy(meta_vmem, meta_smem, meta_sem).wait()
    _start_expert_copies(meta_smem[0, 0], 0, *bufs)

    r2 = jax.lax.broadcasted_iota(jnp.int32, (E, E), 0)
    c2 = jax.lax.broadcasted_iota(jnp.int32, (E, E), 1)
    lower_tri = (r2 <= c2).astype(jnp.float32)
    pos = jax.lax.dot_general(
        act.astype(jnp.float32), lower_tri, (((1,), (0,)), ((), ())),
        preferred_element_type=jnp.float32)
    na_f = jnp.max(pos, axis=1, keepdims=True)
    rowi = jax.lax.broadcasted_iota(jnp.int32, (E + 1, E), 0).astype(jnp.float32)
    match = (jnp.broadcast_to(pos, (E + 1, E)) == rowi + 1.0) \
        & jnp.broadcast_to(act, (E + 1, E))
    cole = jax.lax.broadcasted_iota(jnp.int32, (E + 1, E), 1).astype(jnp.float32)
    vals = jnp.sum(jnp.where(match, cole, 0.0), axis=1, keepdims=True)
    rows1 = jax.lax.broadcasted_iota(jnp.int32, (E + 1, 1), 0).astype(jnp.float32)
    meta = jnp.where(rows1 == float(E), na_f, vals)
    meta_vmem[...] = meta.astype(jnp.int32)
    pltpu.make_async_copy(meta_vmem, meta_smem, meta_sem).start()
    pltpu.make_async_copy(meta_vmem, meta_smem, meta_sem).wait()

    na = meta_smem[E, 0]
    out_ref[...] = x

    @pl.when(na > 1)
    def _pf1():
        _start_expert_copies(meta_smem[1, 0], 1, *bufs)

    def loop_body(s, carry):
        e = meta_smem[s, 0]
        sl = jax.lax.rem(s, 2)
        copies = _expert_copies(e, sl, *bufs)
        t_bf = t_scr[...].astype(jnp.bfloat16)
        copies[0].wait()
        copies[4].wait()
        b1 = b1buf[sl]
        h0 = jax.lax.dot_general(
            t_bf, w1buf[sl, pl.ds(0, F)].astype(jnp.bfloat16),
            (((1,), (1,)), ((), ())),
            preferred_element_type=jnp.float32) + b1[:, :F]
        copies[1].wait()
        h1 = jax.lax.dot_general(
            t_bf, w1buf[sl, pl.ds(F, F)].astype(jnp.bfloat16),
            (((1,), (1,)), ((), ())),
            preferred_element_type=jnp.float32) + b1[:, F:]
        a = (h0 * jax.nn.sigmoid(1.702 * h0) * (h1 + 1.0)).astype(jnp.bfloat16)
        copies[2].wait()
        o_lo = jax.lax.dot_general(
            a, w2buf[sl, pl.ds(0, half)].astype(jnp.bfloat16),
            (((1,), (1,)), ((), ())),
            preferred_element_type=jnp.float32)
        copies[3].wait()
        o_hi = jax.lax.dot_general(
            a, w2buf[sl, pl.ds(half, half)].astype(jnp.bfloat16),
            (((1,), (1,)), ((), ())),
            preferred_element_type=jnp.float32)
        copies[5].wait()
        o = jnp.concatenate([o_lo, o_hi], axis=1) + b2buf[sl]
        w_all = w_scr[...]
        ecol = jax.lax.broadcasted_iota(jnp.int32, w_all.shape, 1)
        wcol = jnp.sum(jnp.where(ecol == e, w_all, 0.0), axis=1, keepdims=True)
        out_ref[...] += o * wcol

        @pl.when(s + 2 < na)
        def _pf():
            _start_expert_copies(meta_smem[s + 2, 0], sl, *bufs)

        return carry

    jax.lax.fori_loop(0, na, loop_body, 0)


def kernel(x, scale, gate_w, gate_b, mlp1_weight, mlp1_bias, mlp2_weight, mlp2_bias):
    B, D = x.shape
    E, twoF, _ = mlp1_weight.shape
    F = twoF // 2

    scale2 = scale.reshape(1, D)
    gate_b2 = gate_b.reshape(1, E)
    b1_3d = mlp1_bias.reshape(E, 1, twoF)
    b2_3d = mlp2_bias.reshape(E, 1, D)

    out = pl.pallas_call(
        lambda *refs: _body(E, F, B, D, *refs),
        in_specs=[
            pl.BlockSpec((B, D), lambda: (0, 0)),
            pl.BlockSpec((1, D), lambda: (0, 0)),
            pl.BlockSpec((E, D), lambda: (0, 0)),
            pl.BlockSpec((1, E), lambda: (0, 0)),
            pl.BlockSpec(memory_space=pltpu.MemorySpace.HBM),
            pl.BlockSpec(memory_space=pltpu.MemorySpace.HBM),
            pl.BlockSpec(memory_space=pltpu.MemorySpace.HBM),
            pl.BlockSpec(memory_space=pltpu.MemorySpace.HBM),
        ],
        out_specs=pl.BlockSpec((B, D), lambda: (0, 0)),
        out_shape=jax.ShapeDtypeStruct((B, D), jnp.float32),
        scratch_shapes=[
            pltpu.VMEM((B, D), jnp.float32),
            pltpu.VMEM((B, E), jnp.float32),
            pltpu.VMEM((E + 1, 1), jnp.int32),
            pltpu.SMEM((E + 1, 1), jnp.int32),
            pltpu.VMEM((2, twoF, D), jnp.float32),
            pltpu.VMEM((2, 1, twoF), jnp.float32),
            pltpu.VMEM((2, D, F), jnp.float32),
            pltpu.VMEM((2, 1, D), jnp.float32),
            pltpu.SemaphoreType.DMA,
            pltpu.SemaphoreType.DMA((2, 6)),
        ],
    )(x, scale2, gate_w, gate_b2, mlp1_weight, b1_3d, mlp2_weight, b2_3d)
    return out


# speedup vs baseline: 1.1011x; 1.1011x over previous
import jax
import jax.numpy as jnp
from jax.experimental import pallas as pl
from jax.experimental.pallas import tpu as pltpu


def _start_expert_copies(e, s, w1_hbm, b1_hbm, w2_hbm, b2_hbm,
                         w1buf, b1buf, w2buf, b2buf, copy_sems):
    pltpu.make_async_copy(w1_hbm.at[e], w1buf.at[s], copy_sems.at[s, 0]).start()
    pltpu.make_async_copy(w2_hbm.at[e], w2buf.at[s], copy_sems.at[s, 1]).start()
    pltpu.make_async_copy(b1_hbm.at[e], b1buf.at[s], copy_sems.at[s, 2]).start()
    pltpu.make_async_copy(b2_hbm.at[e], b2buf.at[s], copy_sems.at[s, 3]).start()


def _wait_expert_copies(e, s, w1_hbm, b1_hbm, w2_hbm, b2_hbm,
                        w1buf, b1buf, w2buf, b2buf, copy_sems):
    pltpu.make_async_copy(w1_hbm.at[e], w1buf.at[s], copy_sems.at[s, 0]).wait()
    pltpu.make_async_copy(w2_hbm.at[e], w2buf.at[s], copy_sems.at[s, 1]).wait()
    pltpu.make_async_copy(b1_hbm.at[e], b1buf.at[s], copy_sems.at[s, 2]).wait()
    pltpu.make_async_copy(b2_hbm.at[e], b2buf.at[s], copy_sems.at[s, 3]).wait()


def _body(E, F, B, D,
          x_ref, scale_ref, gw_ref, gb_ref, w1_hbm, b1_hbm, w2_hbm, b2_hbm,
          out_ref,
          t_scr, w_scr, meta_vmem, meta_smem,
          w1buf, b1buf, w2buf, b2buf, meta_sem, copy_sems):
    bufs = (w1_hbm, b1_hbm, w2_hbm, b2_hbm, w1buf, b1buf, w2buf, b2buf,
            copy_sems)

    x = x_ref[...]
    t = x * jax.lax.rsqrt(jnp.mean(x * x, axis=-1, keepdims=True) + 1e-5)
    t = t * scale_ref[...]
    t_scr[...] = t
    g = jax.lax.dot_general(
        t, gw_ref[...], (((1,), (1,)), ((), ())),
        preferred_element_type=jnp.float32) + gb_ref[...]

    col = jax.lax.broadcasted_iota(jnp.int32, g.shape, 1)
    v1 = jnp.max(g, axis=-1, keepdims=True)
    e1 = jnp.min(jnp.where(g == v1, col, E), axis=-1, keepdims=True)
    first1 = (col == e1)
    g2 = jnp.where(first1, -1e30, g)
    v2 = jnp.max(g2, axis=-1, keepdims=True)
    e2 = jnp.min(jnp.where(g2 == v2, col, E), axis=-1, keepdims=True)
    first2 = (col == e2)
    p1 = jax.nn.sigmoid(v1 - v2)
    wmat = jnp.where(first1, p1, 0.0) + jnp.where(first2, 1.0 - p1, 0.0)
    w_scr[...] = wmat

    act = (jnp.max(wmat, axis=0, keepdims=True) > 0.0)
    coli = jax.lax.broadcasted_iota(jnp.int32, (1, E), 1)
    e0v = jnp.min(jnp.where(act, coli, E), axis=1, keepdims=True)
    meta_vmem[...] = jnp.broadcast_to(e0v, (E + 1, 1)).astype(jnp.int32)
    pltpu.make_async_copy(meta_vmem, meta_smem, meta_sem).start()
    pltpu.make_async_copy(meta_vmem, meta_smem, meta_sem).wait()
    _start_expert_copies(meta_smem[0, 0], 0, *bufs)

    r2 = jax.lax.broadcasted_iota(jnp.int32, (E, E), 0)
    c2 = jax.lax.broadcasted_iota(jnp.int32, (E, E), 1)
    lower_tri = (r2 <= c2).astype(jnp.float32)
    pos = jax.lax.dot_general(
        act.astype(jnp.float32), lower_tri, (((1,), (0,)), ((), ())),
        preferred_element_type=jnp.float32)
    na_f = jnp.max(pos, axis=1, keepdims=True)
    rowi = jax.lax.broadcasted_iota(jnp.int32, (E + 1, E), 0).astype(jnp.float32)
    match = (jnp.broadcast_to(pos, (E + 1, E)) == rowi + 1.0) \
        & jnp.broadcast_to(act, (E + 1, E))
    cole = jax.lax.broadcasted_iota(jnp.int32, (E + 1, E), 1).astype(jnp.float32)
    vals = jnp.sum(jnp.where(match, cole, 0.0), axis=1, keepdims=True)
    rows1 = jax.lax.broadcasted_iota(jnp.int32, (E + 1, 1), 0).astype(jnp.float32)
    meta = jnp.where(rows1 == float(E), na_f, vals)
    meta_vmem[...] = meta.astype(jnp.int32)
    pltpu.make_async_copy(meta_vmem, meta_smem, meta_sem).start()
    pltpu.make_async_copy(meta_vmem, meta_smem, meta_sem).wait()

    na = meta_smem[E, 0]
    out_ref[...] = x

    @pl.when(na > 1)
    def _pf1():
        _start_expert_copies(meta_smem[1, 0], 1, *bufs)

    def loop_body(s, carry):
        e = meta_smem[s, 0]
        sl = jax.lax.rem(s, 2)
        _wait_expert_copies(e, sl, *bufs)
        h = jax.lax.dot_general(
            t_scr[...].astype(jnp.bfloat16), w1buf[sl].astype(jnp.bfloat16),
            (((1,), (1,)), ((), ())),
            preferred_element_type=jnp.float32) + b1buf[sl]
        x_glu = h[:, :F]
        x_lin = h[:, F:]
        a = x_glu * jax.nn.sigmoid(1.702 * x_glu) * (x_lin + 1.0)
        o = jax.lax.dot_general(
            a.astype(jnp.bfloat16), w2buf[sl].astype(jnp.bfloat16),
            (((1,), (1,)), ((), ())),
            preferred_element_type=jnp.float32) + b2buf[sl]
        w_all = w_scr[...]
        ecol = jax.lax.broadcasted_iota(jnp.int32, w_all.shape, 1)
        wcol = jnp.sum(jnp.where(ecol == e, w_all, 0.0), axis=1, keepdims=True)
        out_ref[...] += o * wcol

        @pl.when(s + 2 < na)
        def _pf():
            _start_expert_copies(meta_smem[s + 2, 0], sl, *bufs)

        return carry

    jax.lax.fori_loop(0, na, loop_body, 0)


def kernel(x, scale, gate_w, gate_b, mlp1_weight, mlp1_bias, mlp2_weight, mlp2_bias):
    B, D = x.shape
    E, twoF, _ = mlp1_weight.shape
    F = twoF // 2

    scale2 = scale.reshape(1, D)
    gate_b2 = gate_b.reshape(1, E)
    b1_3d = mlp1_bias.reshape(E, 1, twoF)
    b2_3d = mlp2_bias.reshape(E, 1, D)

    out = pl.pallas_call(
        lambda *refs: _body(E, F, B, D, *refs),
        in_specs=[
            pl.BlockSpec((B, D), lambda: (0, 0)),
            pl.BlockSpec((1, D), lambda: (0, 0)),
            pl.BlockSpec((E, D), lambda: (0, 0)),
            pl.BlockSpec((1, E), lambda: (0, 0)),
            pl.BlockSpec(memory_space=pltpu.MemorySpace.HBM),
            pl.BlockSpec(memory_space=pltpu.MemorySpace.HBM),
            pl.BlockSpec(memory_space=pltpu.MemorySpace.HBM),
            pl.BlockSpec(memory_space=pltpu.MemorySpace.HBM),
        ],
        out_specs=pl.BlockSpec((B, D), lambda: (0, 0)),
        out_shape=jax.ShapeDtypeStruct((B, D), jnp.float32),
        scratch_shapes=[
            pltpu.VMEM((B, D), jnp.float32),
            pltpu.VMEM((B, E), jnp.float32),
            pltpu.VMEM((E + 1, 1), jnp.int32),
            pltpu.SMEM((E + 1, 1), jnp.int32),
            pltpu.VMEM((2, twoF, D), jnp.float32),
            pltpu.VMEM((2, 1, twoF), jnp.float32),
            pltpu.VMEM((2, D, F), jnp.float32),
            pltpu.VMEM((2, 1, D), jnp.float32),
            pltpu.SemaphoreType.DMA,
            pltpu.SemaphoreType.DMA((2, 4)),
        ],
    )(x, scale2, gate_w, gate_b2, mlp1_weight, b1_3d, mlp2_weight, b2_3d)
    return out
